# baseline (device time: 65547 ns/iter reference)
import jax
import jax.numpy as jnp
from jax import lax
from jax.experimental import pallas as pl
from jax.experimental.pallas import tpu as pltpu

N_DEV = 8
B, SQ, SKV, DH = 2, 128, 128, 64
H_LOC = 4
D_MODEL = 512
D_HID = H_LOC * DH


def kernel(x, Wq, K_ext, V_ext, Wo):
    idx = lax.axis_index("i")
    wq_l = lax.dynamic_slice_in_dim(Wq, idx * D_HID, D_HID, axis=1)
    wo_l = lax.dynamic_slice_in_dim(Wo, idx * D_HID, D_HID, axis=0)

    def body(x_ref, wq_ref, k_ref, v_ref, wo_ref, out_ref,
             comm_ref, send_sems, recv_sems):
        my = lax.axis_index("i")
        right = lax.rem(my + 1, N_DEV)
        left = lax.rem(my + N_DEV - 1, N_DEV)

        barrier = pltpu.get_barrier_semaphore()
        for nbr in (left, right):
            pl.semaphore_signal(
                barrier, inc=1,
                device_id=(nbr,), device_id_type=pl.DeviceIdType.MESH,
            )
        pl.semaphore_wait(barrier, 2)

        for b in range(B):
            qb = jnp.dot(x_ref[b], wq_ref[:],
                         preferred_element_type=jnp.float32)
            pb = jnp.zeros((SQ, D_MODEL), jnp.float32)
            for h in range(H_LOC):
                q = qb[:, h * DH:(h + 1) * DH]
                k = k_ref[b, :, h, :]
                v = v_ref[b, :, h, :]
                s = lax.dot_general(
                    q, k, (((1,), (1,)), ((), ())),
                    preferred_element_type=jnp.float32) * 0.125
                m = jnp.max(s, axis=-1, keepdims=True)
                w = jnp.exp(s - m)
                w = w / jnp.sum(w, axis=-1, keepdims=True)
                ctx = jnp.dot(w, v, preferred_element_type=jnp.float32)
                pb = pb + jnp.dot(ctx, wo_ref[h * DH:(h + 1) * DH, :],
                                  preferred_element_type=jnp.float32)
            comm_ref[0, b] = pb
            out_ref[b] = pb

        for hop in range(N_DEV - 1):
            rdma = pltpu.make_async_remote_copy(
                src_ref=comm_ref.at[hop],
                dst_ref=comm_ref.at[hop + 1],
                send_sem=send_sems.at[hop],
                recv_sem=recv_sems.at[hop],
                device_id=(right,),
                device_id_type=pl.DeviceIdType.MESH,
            )
            rdma.start()
            rdma.wait()
            out_ref[:] = out_ref[:] + comm_ref[hop + 1]

    return pl.pallas_call(
        body,
        out_shape=jax.ShapeDtypeStruct((B, SQ, D_MODEL), jnp.float32),
        in_specs=[pl.BlockSpec(memory_space=pltpu.VMEM)] * 5,
        out_specs=pl.BlockSpec(memory_space=pltpu.VMEM),
        scratch_shapes=[
            pltpu.VMEM((N_DEV, B, SQ, D_MODEL), jnp.float32),
            pltpu.SemaphoreType.DMA((N_DEV - 1,)),
            pltpu.SemaphoreType.DMA((N_DEV - 1,)),
        ],
        compiler_params=pltpu.CompilerParams(collective_id=0),
    )(x, wq_l, K_ext, V_ext, wo_l)


# device time: 24471 ns/iter; 2.6786x vs baseline; 2.6786x over previous
import jax
import jax.numpy as jnp
from jax import lax
from jax.experimental import pallas as pl
from jax.experimental.pallas import tpu as pltpu

N_DEV = 8
B, SQ, SKV, DH = 2, 128, 128, 64
H_LOC = 4
D_MODEL = 512
D_HID = H_LOC * DH
ROWS = B * SQ
CHUNK = ROWS // N_DEV


def kernel(x, Wq, K_ext, V_ext, Wo):
    idx = lax.axis_index("i")
    wq_l = lax.dynamic_slice_in_dim(Wq, idx * D_HID, D_HID, axis=1)
    wo_l = lax.dynamic_slice_in_dim(Wo, idx * D_HID, D_HID, axis=0)

    def body(x_ref, wq_ref, k_ref, v_ref, wo_ref, out_ref,
             psrc_ref, rs_ref, sbuf_ref, send_sems, recv_sems):
        my = lax.axis_index("i")

        barrier = pltpu.get_barrier_semaphore()
        for d in range(N_DEV):
            @pl.when(d != my)
            def _():
                pl.semaphore_signal(
                    barrier, inc=1,
                    device_id=(d,), device_id_type=pl.DeviceIdType.MESH,
                )
        pl.semaphore_wait(barrier, N_DEV - 1)

        for b in range(B):
            qb = jnp.dot(x_ref[b], wq_ref[:],
                         preferred_element_type=jnp.float32)
            pb = jnp.zeros((SQ, D_MODEL), jnp.float32)
            for h in range(H_LOC):
                q = qb[:, h * DH:(h + 1) * DH]
                k = k_ref[b, :, h, :]
                v = v_ref[b, :, h, :]
                s = lax.dot_general(
                    q, k, (((1,), (1,)), ((), ())),
                    preferred_element_type=jnp.float32) * 0.125
                m = jnp.max(s, axis=-1, keepdims=True)
                w = jnp.exp(s - m)
                w = w / jnp.sum(w, axis=-1, keepdims=True)
                ctx = jnp.dot(w, v, preferred_element_type=jnp.float32)
                pb = pb + jnp.dot(ctx, wo_ref[h * DH:(h + 1) * DH, :],
                                  preferred_element_type=jnp.float32)
            psrc_ref[b * SQ:(b + 1) * SQ, :] = pb

        for d in range(N_DEV):
            @pl.when(d != my)
            def _():
                rdma = pltpu.make_async_remote_copy(
                    src_ref=psrc_ref.at[pl.ds(d * CHUNK, CHUNK), :],
                    dst_ref=rs_ref.at[my],
                    send_sem=send_sems.at[d],
                    recv_sem=recv_sems.at[0],
                    device_id=(d,),
                    device_id_type=pl.DeviceIdType.MESH,
                )
                rdma.start()

        rs_ref[my] = psrc_ref[pl.ds(my * CHUNK, CHUNK), :]

        for d in range(N_DEV):
            @pl.when(d != my)
            def _():
                pltpu.make_async_remote_copy(
                    src_ref=psrc_ref.at[pl.ds(0, CHUNK), :],
                    dst_ref=rs_ref.at[d],
                    send_sem=send_sems.at[d],
                    recv_sem=recv_sems.at[0],
                    device_id=(d,),
                    device_id_type=pl.DeviceIdType.MESH,
                ).wait_recv()

        s = rs_ref[0]
        for d in range(1, N_DEV):
            s = s + rs_ref[d]
        sbuf_ref[:, :] = s
        out_ref[pl.ds(my * CHUNK, CHUNK), :] = s

        for d in range(N_DEV):
            @pl.when(d != my)
            def _():
                rdma = pltpu.make_async_remote_copy(
                    src_ref=sbuf_ref,
                    dst_ref=out_ref.at[pl.ds(my * CHUNK, CHUNK), :],
                    send_sem=send_sems.at[N_DEV + d],
                    recv_sem=recv_sems.at[1],
                    device_id=(d,),
                    device_id_type=pl.DeviceIdType.MESH,
                )
                rdma.start()

        for d in range(N_DEV):
            @pl.when(d != my)
            def _():
                pltpu.make_async_remote_copy(
                    src_ref=sbuf_ref,
                    dst_ref=out_ref.at[pl.ds(d * CHUNK, CHUNK), :],
                    send_sem=send_sems.at[N_DEV + d],
                    recv_sem=recv_sems.at[1],
                    device_id=(d,),
                    device_id_type=pl.DeviceIdType.MESH,
                ).wait_recv()

        for d in range(N_DEV):
            @pl.when(d != my)
            def _():
                pltpu.make_async_remote_copy(
                    src_ref=psrc_ref.at[pl.ds(d * CHUNK, CHUNK), :],
                    dst_ref=rs_ref.at[my],
                    send_sem=send_sems.at[d],
                    recv_sem=recv_sems.at[0],
                    device_id=(d,),
                    device_id_type=pl.DeviceIdType.MESH,
                ).wait_send()
                pltpu.make_async_remote_copy(
                    src_ref=sbuf_ref,
                    dst_ref=out_ref.at[pl.ds(my * CHUNK, CHUNK), :],
                    send_sem=send_sems.at[N_DEV + d],
                    recv_sem=recv_sems.at[1],
                    device_id=(d,),
                    device_id_type=pl.DeviceIdType.MESH,
                ).wait_send()

    out_flat = pl.pallas_call(
        body,
        out_shape=jax.ShapeDtypeStruct((ROWS, D_MODEL), jnp.float32),
        in_specs=[pl.BlockSpec(memory_space=pltpu.VMEM)] * 5,
        out_specs=pl.BlockSpec(memory_space=pltpu.VMEM),
        scratch_shapes=[
            pltpu.VMEM((ROWS, D_MODEL), jnp.float32),
            pltpu.VMEM((N_DEV, CHUNK, D_MODEL), jnp.float32),
            pltpu.VMEM((CHUNK, D_MODEL), jnp.float32),
            pltpu.SemaphoreType.DMA((2 * N_DEV,)),
            pltpu.SemaphoreType.DMA((2,)),
        ],
        compiler_params=pltpu.CompilerParams(collective_id=0),
    )(x, wq_l, K_ext, V_ext, wo_l)
    return out_flat.reshape(B, SQ, D_MODEL)


# device time: 22596 ns/iter; 2.9008x vs baseline; 1.0830x over previous
import jax
import jax.numpy as jnp
from jax import lax
from jax.experimental import pallas as pl
from jax.experimental.pallas import tpu as pltpu

N_DEV = 8
B, SQ, SKV, DH = 2, 128, 128, 64
H_LOC = 4
D_MODEL = 512
D_HID = H_LOC * DH
ROWS = B * SQ
CHUNK = ROWS // N_DEV


def kernel(x, Wq, K_ext, V_ext, Wo):
    idx = lax.axis_index("i")
    wq_l = lax.dynamic_slice_in_dim(Wq, idx * D_HID, D_HID, axis=1)

    def body(x_ref, wq_ref, k_ref, v_ref, wo_ref, out_ref,
             ctx_ref, rs_ref, sbuf_ref, send_sems, recv_sems):
        my = lax.axis_index("i")

        barrier = pltpu.get_barrier_semaphore()
        for d in range(N_DEV):
            @pl.when(d != my)
            def _():
                pl.semaphore_signal(
                    barrier, inc=1,
                    device_id=(d,), device_id_type=pl.DeviceIdType.MESH,
                )
        pl.semaphore_wait(barrier, N_DEV - 1)

        def attn_batch(b):
            qb = jnp.dot(x_ref[b], wq_ref[:],
                         preferred_element_type=jnp.float32)
            heads = []
            for h in range(H_LOC):
                q = qb[:, h * DH:(h + 1) * DH]
                k = k_ref[b, :, h, :]
                v = v_ref[b, :, h, :]
                s = lax.dot_general(
                    q, k, (((1,), (1,)), ((), ())),
                    preferred_element_type=jnp.float32) * 0.125
                m = jnp.max(s, axis=-1, keepdims=True)
                w = jnp.exp(s - m)
                w = w / jnp.sum(w, axis=-1, keepdims=True)
                heads.append(jnp.dot(w, v, preferred_element_type=jnp.float32))
            return jnp.concatenate(heads, axis=1)

        chunks_per_b = SQ // CHUNK
        for b in range(B):
            ctx_ref[b * SQ:(b + 1) * SQ, :] = attn_batch(b)
            for d in range(b * chunks_per_b, (b + 1) * chunks_per_b):
                @pl.when(d != my)
                def _():
                    pltpu.make_async_remote_copy(
                        src_ref=ctx_ref.at[pl.ds(d * CHUNK, CHUNK), :],
                        dst_ref=rs_ref.at[my],
                        send_sem=send_sems.at[d],
                        recv_sem=recv_sems.at[0],
                        device_id=(d,),
                        device_id_type=pl.DeviceIdType.MESH,
                    ).start()

        rs_ref[my] = ctx_ref[pl.ds(my * CHUNK, CHUNK), :]

        for d in range(N_DEV):
            @pl.when(d != my)
            def _():
                pltpu.make_async_remote_copy(
                    src_ref=ctx_ref.at[pl.ds(0, CHUNK), :],
                    dst_ref=rs_ref.at[d],
                    send_sem=send_sems.at[d],
                    recv_sem=recv_sems.at[0],
                    device_id=(d,),
                    device_id_type=pl.DeviceIdType.MESH,
                ).wait_recv()

        acc = jnp.dot(rs_ref[0], wo_ref[0:D_HID, :],
                      preferred_element_type=jnp.float32)
        for d in range(1, N_DEV):
            acc = acc + jnp.dot(rs_ref[d],
                                wo_ref[d * D_HID:(d + 1) * D_HID, :],
                                preferred_element_type=jnp.float32)
        sbuf_ref[:, :] = acc
        out_ref[pl.ds(my * CHUNK, CHUNK), :] = acc

        for d in range(N_DEV):
            @pl.when(d != my)
            def _():
                pltpu.make_async_remote_copy(
                    src_ref=sbuf_ref,
                    dst_ref=out_ref.at[pl.ds(my * CHUNK, CHUNK), :],
                    send_sem=send_sems.at[N_DEV + d],
                    recv_sem=recv_sems.at[1],
                    device_id=(d,),
                    device_id_type=pl.DeviceIdType.MESH,
                ).start()

        for d in range(N_DEV):
            @pl.when(d != my)
            def _():
                pltpu.make_async_remote_copy(
                    src_ref=sbuf_ref,
                    dst_ref=out_ref.at[pl.ds(d * CHUNK, CHUNK), :],
                    send_sem=send_sems.at[N_DEV + d],
                    recv_sem=recv_sems.at[1],
                    device_id=(d,),
                    device_id_type=pl.DeviceIdType.MESH,
                ).wait_recv()

        for d in range(N_DEV):
            @pl.when(d != my)
            def _():
                pltpu.make_async_remote_copy(
                    src_ref=ctx_ref.at[pl.ds(d * CHUNK, CHUNK), :],
                    dst_ref=rs_ref.at[my],
                    send_sem=send_sems.at[d],
                    recv_sem=recv_sems.at[0],
                    device_id=(d,),
                    device_id_type=pl.DeviceIdType.MESH,
                ).wait_send()
                pltpu.make_async_remote_copy(
                    src_ref=sbuf_ref,
                    dst_ref=out_ref.at[pl.ds(my * CHUNK, CHUNK), :],
                    send_sem=send_sems.at[N_DEV + d],
                    recv_sem=recv_sems.at[1],
                    device_id=(d,),
                    device_id_type=pl.DeviceIdType.MESH,
                ).wait_send()

    out_flat = pl.pallas_call(
        body,
        out_shape=jax.ShapeDtypeStruct((ROWS, D_MODEL), jnp.float32),
        in_specs=[pl.BlockSpec(memory_space=pltpu.VMEM)] * 5,
        out_specs=pl.BlockSpec(memory_space=pltpu.VMEM),
        scratch_shapes=[
            pltpu.VMEM((ROWS, D_HID), jnp.float32),
            pltpu.VMEM((N_DEV, CHUNK, D_HID), jnp.float32),
            pltpu.VMEM((CHUNK, D_MODEL), jnp.float32),
            pltpu.SemaphoreType.DMA((2 * N_DEV,)),
            pltpu.SemaphoreType.DMA((2,)),
        ],
        compiler_params=pltpu.CompilerParams(collective_id=0),
    )(x, wq_l, K_ext, V_ext, Wo)
    return out_flat.reshape(B, SQ, D_MODEL)


# device time: 10027 ns/iter; 6.5370x vs baseline; 2.2535x over previous
import os

import jax
import jax.numpy as jnp
from jax import lax

_NOCOMM = os.environ.get("NOCOMM") == "1"
from jax.experimental import pallas as pl
from jax.experimental.pallas import tpu as pltpu

N_DEV = 8
B, SQ, SKV, DH = 2, 128, 128, 64
H_LOC = 4
D_MODEL = 512
D_HID = H_LOC * DH
ROWS = B * SQ
CHUNK = ROWS // N_DEV


def kernel(x, Wq, K_ext, V_ext, Wo):
    idx = lax.axis_index("i")
    wq_l = lax.dynamic_slice_in_dim(Wq, idx * D_HID, D_HID, axis=1)

    def body(x_ref, wq_ref, k_ref, v_ref, wo_ref, out_ref,
             ctx_ref, rs_ref, sbuf_ref, send_sems, recv_sems):
        my = lax.axis_index("i")

        if not _NOCOMM:
            barrier = pltpu.get_barrier_semaphore()
            for d in range(N_DEV):
                @pl.when(d != my)
                def _():
                    pl.semaphore_signal(
                        barrier, inc=1,
                        device_id=(d,), device_id_type=pl.DeviceIdType.MESH,
                    )
            pl.semaphore_wait(barrier, N_DEV - 1)

        def attn_batch(b):
            qb = jnp.dot(x_ref[b], wq_ref[:],
                         preferred_element_type=jnp.float32)
            heads = []
            for h in range(H_LOC):
                q = qb[:, h * DH:(h + 1) * DH]
                k = k_ref[b, :, h, :]
                v = v_ref[b, :, h, :]
                s = lax.dot_general(
                    q, k, (((1,), (1,)), ((), ())),
                    preferred_element_type=jnp.float32) * 0.125
                m = jnp.max(s, axis=-1, keepdims=True)
                w = jnp.exp(s - m)
                w = w / jnp.sum(w, axis=-1, keepdims=True)
                heads.append(jnp.dot(w, v, preferred_element_type=jnp.float32))
            return jnp.concatenate(heads, axis=1)

        chunks_per_b = SQ // CHUNK
        for b in range(B):
            ctx_ref[b * SQ:(b + 1) * SQ, :] = attn_batch(b)
            for d in range(b * chunks_per_b if not _NOCOMM else 0,
                           (b + 1) * chunks_per_b if not _NOCOMM else 0):
                @pl.when(d != my)
                def _():
                    pltpu.make_async_remote_copy(
                        src_ref=ctx_ref.at[pl.ds(d * CHUNK, CHUNK), :],
                        dst_ref=rs_ref.at[my],
                        send_sem=send_sems.at[d],
                        recv_sem=recv_sems.at[0],
                        device_id=(d,),
                        device_id_type=pl.DeviceIdType.MESH,
                    ).start()

        rs_ref[my] = ctx_ref[pl.ds(my * CHUNK, CHUNK), :]

        for d in range(N_DEV if not _NOCOMM else 0):
            @pl.when(d != my)
            def _():
                pltpu.make_async_remote_copy(
                    src_ref=ctx_ref.at[pl.ds(0, CHUNK), :],
                    dst_ref=rs_ref.at[d],
                    send_sem=send_sems.at[d],
                    recv_sem=recv_sems.at[0],
                    device_id=(d,),
                    device_id_type=pl.DeviceIdType.MESH,
                ).wait_recv()

        acc = jnp.dot(rs_ref[0], wo_ref[0:D_HID, :],
                      preferred_element_type=jnp.float32)
        for d in range(1, N_DEV):
            acc = acc + jnp.dot(rs_ref[d],
                                wo_ref[d * D_HID:(d + 1) * D_HID, :],
                                preferred_element_type=jnp.float32)
        sbuf_ref[:, :] = acc
        out_ref[pl.ds(my * CHUNK, CHUNK), :] = acc

        for d in range(N_DEV if not _NOCOMM else 0):
            @pl.when(d != my)
            def _():
                pltpu.make_async_remote_copy(
                    src_ref=sbuf_ref,
                    dst_ref=out_ref.at[pl.ds(my * CHUNK, CHUNK), :],
                    send_sem=send_sems.at[N_DEV + d],
                    recv_sem=recv_sems.at[1],
                    device_id=(d,),
                    device_id_type=pl.DeviceIdType.MESH,
                ).start()

        for d in range(N_DEV if not _NOCOMM else 0):
            @pl.when(d != my)
            def _():
                pltpu.make_async_remote_copy(
                    src_ref=sbuf_ref,
                    dst_ref=out_ref.at[pl.ds(d * CHUNK, CHUNK), :],
                    send_sem=send_sems.at[N_DEV + d],
                    recv_sem=recv_sems.at[1],
                    device_id=(d,),
                    device_id_type=pl.DeviceIdType.MESH,
                ).wait_recv()

        for d in range(N_DEV if not _NOCOMM else 0):
            @pl.when(d != my)
            def _():
                pltpu.make_async_remote_copy(
                    src_ref=ctx_ref.at[pl.ds(d * CHUNK, CHUNK), :],
                    dst_ref=rs_ref.at[my],
                    send_sem=send_sems.at[d],
                    recv_sem=recv_sems.at[0],
                    device_id=(d,),
                    device_id_type=pl.DeviceIdType.MESH,
                ).wait_send()
                pltpu.make_async_remote_copy(
                    src_ref=sbuf_ref,
                    dst_ref=out_ref.at[pl.ds(my * CHUNK, CHUNK), :],
                    send_sem=send_sems.at[N_DEV + d],
                    recv_sem=recv_sems.at[1],
                    device_id=(d,),
                    device_id_type=pl.DeviceIdType.MESH,
                ).wait_send()

    out_flat = pl.pallas_call(
        body,
        out_shape=jax.ShapeDtypeStruct((ROWS, D_MODEL), jnp.float32),
        in_specs=[pl.BlockSpec(memory_space=pltpu.VMEM)] * 5,
        out_specs=pl.BlockSpec(memory_space=pltpu.VMEM),
        scratch_shapes=[
            pltpu.VMEM((ROWS, D_HID), jnp.float32),
            pltpu.VMEM((N_DEV, CHUNK, D_HID), jnp.float32),
            pltpu.VMEM((CHUNK, D_MODEL), jnp.float32),
            pltpu.SemaphoreType.DMA((2 * N_DEV,)),
            pltpu.SemaphoreType.DMA((2,)),
        ],
        compiler_params=(None if _NOCOMM
                         else pltpu.CompilerParams(collective_id=0)),
    )(x, wq_l, K_ext, V_ext, Wo)
    return out_flat.reshape(B, SQ, D_MODEL)


# device time: 7021 ns/iter; 9.3358x vs baseline; 1.4281x over previous
import os

import jax
import jax.numpy as jnp
from jax import lax

_NOCOMM = os.environ.get("NOCOMM") == "1"
_NOCOMPUTE = os.environ.get("NOCOMPUTE") == "1"
from jax.experimental import pallas as pl
from jax.experimental.pallas import tpu as pltpu

N_DEV = 8
B, SQ, SKV, DH = 2, 128, 128, 64
H_LOC = 4
D_MODEL = 512
D_HID = H_LOC * DH
ROWS = B * SQ
CHUNK = ROWS // N_DEV


def kernel(x, Wq, K_ext, V_ext, Wo):
    idx = lax.axis_index("i")
    wq_l = lax.dynamic_slice_in_dim(Wq, idx * D_HID, D_HID, axis=1)

    def body(x_ref, wq_ref, k_ref, v_ref, wo_ref, out_ref,
             ctx_ref, rs_ref, sbuf_ref, send_sems, recv_sems):
        my = lax.axis_index("i")

        if not _NOCOMM:
            barrier = pltpu.get_barrier_semaphore()
            for d in range(N_DEV):
                @pl.when(d != my)
                def _():
                    pl.semaphore_signal(
                        barrier, inc=1,
                        device_id=(d,), device_id_type=pl.DeviceIdType.MESH,
                    )
            pl.semaphore_wait(barrier, N_DEV - 1)

        def attn_batch(b):
            qb = jnp.dot(x_ref[b], wq_ref[:],
                         preferred_element_type=jnp.float32)
            heads = []
            for h in range(H_LOC):
                q = qb[:, h * DH:(h + 1) * DH]
                k = k_ref[b, :, h, :]
                v = v_ref[b, :, h, :]
                s = lax.dot_general(
                    q, k, (((1,), (1,)), ((), ())),
                    preferred_element_type=jnp.float32) * 0.125
                m = jnp.max(s, axis=-1, keepdims=True)
                w = jnp.exp(s - m)
                w = w / jnp.sum(w, axis=-1, keepdims=True)
                heads.append(jnp.dot(w, v, preferred_element_type=jnp.float32))
            return jnp.concatenate(heads, axis=1)

        chunks_per_b = SQ // CHUNK
        for b in range(B):
            if _NOCOMPUTE:
                ctx_ref[b * SQ:(b + 1) * SQ, :] = x_ref[b, :, 0:D_HID]
            else:
                ctx_ref[b * SQ:(b + 1) * SQ, :] = attn_batch(b)
            for d in range(b * chunks_per_b if not _NOCOMM else 0,
                           (b + 1) * chunks_per_b if not _NOCOMM else 0):
                @pl.when(d != my)
                def _():
                    pltpu.make_async_remote_copy(
                        src_ref=ctx_ref.at[pl.ds(d * CHUNK, CHUNK), :],
                        dst_ref=rs_ref.at[my],
                        send_sem=send_sems.at[d],
                        recv_sem=recv_sems.at[0],
                        device_id=(d,),
                        device_id_type=pl.DeviceIdType.MESH,
                    ).start()

        rs_ref[my] = ctx_ref[pl.ds(my * CHUNK, CHUNK), :]

        for d in range(N_DEV if not _NOCOMM else 0):
            @pl.when(d != my)
            def _():
                pltpu.make_async_remote_copy(
                    src_ref=ctx_ref.at[pl.ds(0, CHUNK), :],
                    dst_ref=rs_ref.at[d],
                    send_sem=send_sems.at[d],
                    recv_sem=recv_sems.at[0],
                    device_id=(d,),
                    device_id_type=pl.DeviceIdType.MESH,
                ).wait_recv()

        if _NOCOMPUTE:
            acc = jnp.concatenate([rs_ref[0], rs_ref[1]], axis=1)
        else:
            acc = jnp.dot(rs_ref[0], wo_ref[0:D_HID, :],
                          preferred_element_type=jnp.float32)
            for d in range(1, N_DEV):
                acc = acc + jnp.dot(rs_ref[d],
                                    wo_ref[d * D_HID:(d + 1) * D_HID, :],
                                    preferred_element_type=jnp.float32)
        sbuf_ref[:, :] = acc
        out_ref[pl.ds(my * CHUNK, CHUNK), :] = acc

        for d in range(N_DEV if not _NOCOMM else 0):
            @pl.when(d != my)
            def _():
                pltpu.make_async_remote_copy(
                    src_ref=sbuf_ref,
                    dst_ref=out_ref.at[pl.ds(my * CHUNK, CHUNK), :],
                    send_sem=send_sems.at[N_DEV + d],
                    recv_sem=recv_sems.at[1],
                    device_id=(d,),
                    device_id_type=pl.DeviceIdType.MESH,
                ).start()

        for d in range(N_DEV if not _NOCOMM else 0):
            @pl.when(d != my)
            def _():
                pltpu.make_async_remote_copy(
                    src_ref=sbuf_ref,
                    dst_ref=out_ref.at[pl.ds(d * CHUNK, CHUNK), :],
                    send_sem=send_sems.at[N_DEV + d],
                    recv_sem=recv_sems.at[1],
                    device_id=(d,),
                    device_id_type=pl.DeviceIdType.MESH,
                ).wait_recv()

        for d in range(N_DEV if not _NOCOMM else 0):
            @pl.when(d != my)
            def _():
                pltpu.make_async_remote_copy(
                    src_ref=ctx_ref.at[pl.ds(d * CHUNK, CHUNK), :],
                    dst_ref=rs_ref.at[my],
                    send_sem=send_sems.at[d],
                    recv_sem=recv_sems.at[0],
                    device_id=(d,),
                    device_id_type=pl.DeviceIdType.MESH,
                ).wait_send()
                pltpu.make_async_remote_copy(
                    src_ref=sbuf_ref,
                    dst_ref=out_ref.at[pl.ds(my * CHUNK, CHUNK), :],
                    send_sem=send_sems.at[N_DEV + d],
                    recv_sem=recv_sems.at[1],
                    device_id=(d,),
                    device_id_type=pl.DeviceIdType.MESH,
                ).wait_send()

    out_flat = pl.pallas_call(
        body,
        out_shape=jax.ShapeDtypeStruct((ROWS, D_MODEL), jnp.float32),
        in_specs=[pl.BlockSpec(memory_space=pltpu.VMEM)] * 5,
        out_specs=pl.BlockSpec(memory_space=pltpu.VMEM),
        scratch_shapes=[
            pltpu.VMEM((ROWS, D_HID), jnp.float32),
            pltpu.VMEM((N_DEV, CHUNK, D_HID), jnp.float32),
            pltpu.VMEM((CHUNK, D_MODEL), jnp.float32),
            pltpu.SemaphoreType.DMA((2 * N_DEV,)),
            pltpu.SemaphoreType.DMA((2,)),
        ],
        compiler_params=(None if _NOCOMM
                         else pltpu.CompilerParams(collective_id=0)),
    )(x, wq_l, K_ext, V_ext, Wo)
    return out_flat.reshape(B, SQ, D_MODEL)
